# baseline (speedup 1.0000x reference)
import functools

import jax
import jax.numpy as jnp
from jax import lax
from jax.experimental import pallas as pl
from jax.experimental.pallas import tpu as pltpu
from jax.experimental.pallas import tpu_sc as plsc

N_NODES = 10000
IN_CH = 128
OUT_CH = 128
EPS = 1e-5

NC = 2
NS = 16
NW = NC * NS
K = 128
NSPLIT = 4
N_ACC = 10240
STRIPE = N_ACC // NS


def _sc_mesh():
    return plsc.VectorSubcoreMesh(
        core_axis_name="c", subcore_axis_name="s", num_cores=NC, num_subcores=NS
    )


def _deg_body(n_chunks, dst_hbm, zeros_hbm, deg_hbm, dst_v, zbuf, ones_v,
              obuf, deg_s):
    cid = lax.axis_index("c")
    sid = lax.axis_index("s")
    wid = cid * NS + sid

    pltpu.sync_copy(dst_hbm.at[wid, pl.ds(0, n_chunks)], dst_v)
    pltpu.sync_copy(zeros_hbm, zbuf)
    pltpu.sync_copy(zbuf, deg_s.at[pl.ds(sid * STRIPE, STRIPE)])
    for i in range(K // 16):
        ones_v[pl.ds(i * 16, 16)] = jnp.ones((16,), jnp.float32)
    plsc.subcore_barrier()

    def chunk(j, carry):
        pltpu.sync_copy(ones_v, deg_s.at[dst_v.at[j]], add=True)
        return carry

    lax.fori_loop(0, n_chunks, chunk, 0)
    plsc.subcore_barrier()

    pltpu.sync_copy(deg_s.at[pl.ds(sid * STRIPE, STRIPE)], obuf)
    pltpu.sync_copy(obuf, deg_hbm.at[cid, pl.ds(sid * STRIPE, STRIPE)])


def _deg_kernel(dst3, zeros1d):
    n_chunks = dst3.shape[1] - 2
    return pl.kernel(
        functools.partial(_deg_body, n_chunks),
        out_type=jax.ShapeDtypeStruct((NC, N_ACC), jnp.float32),
        mesh=_sc_mesh(),
        scratch_types=[
            pltpu.VMEM((n_chunks, K), jnp.int32),
            pltpu.VMEM((STRIPE,), jnp.float32),
            pltpu.VMEM((K,), jnp.float32),
            pltpu.VMEM((STRIPE,), jnp.float32),
            pltpu.VMEM_SHARED((N_ACC,), jnp.float32),
        ],
    )(dst3, zeros1d)


def _g_body(x_ref, w_ref, deg_ref, g_ref):
    h = lax.dot_general(
        x_ref[...], w_ref[...], (((1,), (1,)), ((), ())),
        preferred_element_type=jnp.float32)
    deg = 1.0 + deg_ref[0, :] + deg_ref[1, :]
    dinv = lax.rsqrt(deg)
    g_ref[...] = h * dinv[:, None]


def _g_kernel(x_pad, W, deg2):
    nb = 10
    blk = N_ACC // nb
    return pl.pallas_call(
        _g_body,
        grid=(nb,),
        in_specs=[
            pl.BlockSpec((blk, IN_CH), lambda i: (i, 0)),
            pl.BlockSpec((OUT_CH, IN_CH), lambda i: (0, 0)),
            pl.BlockSpec((NC, blk), lambda i: (0, i)),
        ],
        out_specs=pl.BlockSpec((blk, OUT_CH), lambda i: (i, 0)),
        out_shape=jax.ShapeDtypeStruct((N_ACC, OUT_CH), jnp.float32),
    )(x_pad, W, deg2)


def _scatter_body(n_chunks, g_hbm, src_hbm, dst_hbm, zeros_hbm, p_hbm,
                  sidx0, sidx1, didx0, didx1, rows0, rows1, acc_s,
                  sems_g0, sems_g1, sem_s0, sem_s1, sem_d0, sem_d1):
    cid = lax.axis_index("c")
    sid = lax.axis_index("s")
    wid = cid * NS + sid
    src_w = src_hbm.at[wid]
    dst_w = dst_hbm.at[wid]

    def idx_start(hbm_row, buf, sem):
        pltpu.async_copy(hbm_row, buf, sem)

    def idx_wait(hbm_row, buf, sem):
        pltpu.make_async_copy(hbm_row, buf, sem).wait()

    SLICE = K // NSPLIT

    def g_start(sbuf, buf, sems):
        for s in range(NSPLIT):
            pltpu.async_copy(g_hbm.at[sbuf.at[pl.ds(s * SLICE, SLICE)]],
                             buf.at[pl.ds(s * SLICE, SLICE)], sems[s])

    def g_wait(sbuf, buf, sems):
        for s in range(NSPLIT):
            pltpu.make_async_copy(g_hbm.at[sbuf.at[pl.ds(s * SLICE, SLICE)]],
                                  buf.at[pl.ds(s * SLICE, SLICE)],
                                  sems[s]).wait()

    idx_start(src_w.at[0], sidx0, sem_s0)
    idx_start(src_w.at[1], sidx1, sem_s1)
    idx_start(dst_w.at[0], didx0, sem_d0)
    idx_start(dst_w.at[1], didx1, sem_d1)

    pltpu.sync_copy(zeros_hbm, rows0)
    for k in range(STRIPE // K):
        pltpu.sync_copy(rows0, acc_s.at[pl.ds(sid * STRIPE + k * K, K)])
    plsc.subcore_barrier()

    idx_wait(src_w.at[0], sidx0, sem_s0)
    g_start(sidx0, rows0, sems_g0)
    idx_wait(src_w.at[1], sidx1, sem_s1)
    g_start(sidx1, rows1, sems_g1)

    def step(i, carry):
        c0 = 2 * i
        c1 = c0 + 1
        g_wait(sidx0, rows0, sems_g0)
        idx_start(src_w.at[c0 + 2], sidx0, sem_s0)
        idx_wait(dst_w.at[c0], didx0, sem_d0)
        pltpu.sync_copy(rows0, acc_s.at[didx0], add=True)
        idx_wait(src_w.at[c0 + 2], sidx0, sem_s0)
        g_start(sidx0, rows0, sems_g0)
        idx_start(dst_w.at[c0 + 2], didx0, sem_d0)
        g_wait(sidx1, rows1, sems_g1)
        idx_start(src_w.at[c1 + 2], sidx1, sem_s1)
        idx_wait(dst_w.at[c1], didx1, sem_d1)
        pltpu.sync_copy(rows1, acc_s.at[didx1], add=True)
        idx_wait(src_w.at[c1 + 2], sidx1, sem_s1)
        g_start(sidx1, rows1, sems_g1)
        idx_start(dst_w.at[c1 + 2], didx1, sem_d1)
        return carry

    lax.fori_loop(0, n_chunks // 2, step, 0)
    g_wait(sidx0, rows0, sems_g0)
    g_wait(sidx1, rows1, sems_g1)
    idx_wait(dst_w.at[0], didx0, sem_d0)
    idx_wait(dst_w.at[1], didx1, sem_d1)
    plsc.subcore_barrier()

    for k in range(STRIPE // K):
        r0 = sid * STRIPE + k * K
        pltpu.sync_copy(acc_s.at[pl.ds(r0, K)], rows0)
        pltpu.sync_copy(rows0, p_hbm.at[cid, pl.ds(r0, K)])


def _scatter_kernel(g, src3, dst3, zeros2d):
    n_chunks = dst3.shape[1] - 2
    return pl.kernel(
        functools.partial(_scatter_body, n_chunks),
        out_type=jax.ShapeDtypeStruct((NC, N_ACC, OUT_CH), jnp.float32),
        mesh=_sc_mesh(),
        scratch_types=[
            pltpu.VMEM((K,), jnp.int32),
            pltpu.VMEM((K,), jnp.int32),
            pltpu.VMEM((K,), jnp.int32),
            pltpu.VMEM((K,), jnp.int32),
            pltpu.VMEM((K, OUT_CH), jnp.float32),
            pltpu.VMEM((K, OUT_CH), jnp.float32),
            pltpu.VMEM_SHARED((N_ACC, OUT_CH), jnp.float32),
            [pltpu.SemaphoreType.DMA] * NSPLIT,
            [pltpu.SemaphoreType.DMA] * NSPLIT,
            pltpu.SemaphoreType.DMA,
            pltpu.SemaphoreType.DMA,
            pltpu.SemaphoreType.DMA,
            pltpu.SemaphoreType.DMA,
        ],
    )(g, src3, dst3, zeros2d)


def _z_body(blk, p_ref, g_ref, deg_ref, b_ref, z_ref, stats_ref):
    i = pl.program_id(0)
    deg = 1.0 + deg_ref[0, :] + deg_ref[1, :]
    dinv = lax.rsqrt(deg)
    z = (p_ref[0] + p_ref[1] + g_ref[...]) * dinv[:, None] + b_ref[...]
    row = i * blk + lax.broadcasted_iota(jnp.int32, (blk, OUT_CH), 0)
    z = jnp.where(row < N_NODES, z, 0.0)
    z_ref[...] = z

    @pl.when(i == 0)
    def _():
        stats_ref[...] = jnp.zeros_like(stats_ref)

    s1 = jnp.sum(z, axis=0)[None, :]
    s2 = jnp.sum(z * z, axis=0)[None, :]
    pad = jnp.zeros((6, OUT_CH), jnp.float32)
    stats_ref[...] += jnp.concatenate([s1, s2, pad], axis=0)


def _z_kernel(P, g, deg2, b):
    nb = 10
    blk = N_ACC // nb
    return pl.pallas_call(
        functools.partial(_z_body, blk),
        grid=(nb,),
        in_specs=[
            pl.BlockSpec((NC, blk, OUT_CH), lambda i: (0, i, 0)),
            pl.BlockSpec((blk, OUT_CH), lambda i: (i, 0)),
            pl.BlockSpec((NC, blk), lambda i: (0, i)),
            pl.BlockSpec((1, OUT_CH), lambda i: (0, 0)),
        ],
        out_specs=[
            pl.BlockSpec((blk, OUT_CH), lambda i: (i, 0)),
            pl.BlockSpec((8, OUT_CH), lambda i: (0, 0)),
        ],
        out_shape=[
            jax.ShapeDtypeStruct((N_ACC, OUT_CH), jnp.float32),
            jax.ShapeDtypeStruct((8, OUT_CH), jnp.float32),
        ],
    )(P, g, deg2, b)


def _bn_body(z_ref, stats_ref, gamma_ref, beta_ref, o_ref):
    n = jnp.float32(N_NODES)
    mean = stats_ref[0, :] / n
    var = stats_ref[1, :] / n - mean * mean
    scale = gamma_ref[0, :] * lax.rsqrt(var + EPS)
    shift = beta_ref[0, :] - mean * scale
    o_ref[...] = jnp.maximum(z_ref[...] * scale[None, :] + shift[None, :], 0.0)


def _bn_kernel(z, stats, gamma, beta):
    nb = 10
    blk = N_ACC // nb
    return pl.pallas_call(
        _bn_body,
        grid=(nb,),
        in_specs=[
            pl.BlockSpec((blk, OUT_CH), lambda i: (i, 0)),
            pl.BlockSpec((8, OUT_CH), lambda i: (0, 0)),
            pl.BlockSpec((1, OUT_CH), lambda i: (0, 0)),
            pl.BlockSpec((1, OUT_CH), lambda i: (0, 0)),
        ],
        out_specs=pl.BlockSpec((blk, OUT_CH), lambda i: (i, 0)),
        out_shape=jax.ShapeDtypeStruct((N_ACC, OUT_CH), jnp.float32),
    )(z, stats, gamma, beta)


def kernel(x, edge_index, W, b, gamma, beta):
    E = edge_index.shape[1]
    src = edge_index[0].astype(jnp.int32)
    dst = edge_index[1].astype(jnp.int32)

    n_chunks = -(-E // (NW * K))
    n_chunks += n_chunks % 2
    cap = NW * n_chunks * K
    src_p = jnp.concatenate([src, jnp.zeros((cap - E,), jnp.int32)])
    trash = N_NODES + jnp.arange(cap - E, dtype=jnp.int32) % (N_ACC - N_NODES)
    dst_p = jnp.concatenate([dst, trash])
    src3 = jnp.concatenate(
        [src_p.reshape(NW, n_chunks, K),
         jnp.zeros((NW, 2, K), jnp.int32)], axis=1)
    dst3 = jnp.concatenate(
        [dst_p.reshape(NW, n_chunks, K),
         jnp.full((NW, 2, K), N_NODES, jnp.int32)], axis=1)

    zeros1d = jnp.zeros((STRIPE,), jnp.float32)
    zeros2d = jnp.zeros((K, OUT_CH), jnp.float32)
    x_pad = jnp.concatenate(
        [x, jnp.zeros((N_ACC - N_NODES, IN_CH), jnp.float32)])

    deg2 = _deg_kernel(dst3, zeros1d)
    g = _g_kernel(x_pad, W, deg2)
    P = _scatter_kernel(g, src3, dst3, zeros2d)
    z, stats = _z_kernel(P, g, deg2, b.reshape(1, OUT_CH))
    out = _bn_kernel(z, stats, gamma.reshape(1, OUT_CH),
                     beta.reshape(1, OUT_CH))
    return out[:N_NODES]


# speedup vs baseline: 1.2175x; 1.2175x over previous
import functools

import jax
import jax.numpy as jnp
from jax import lax
from jax.experimental import pallas as pl
from jax.experimental.pallas import tpu as pltpu
from jax.experimental.pallas import tpu_sc as plsc

N_NODES = 10000
IN_CH = 128
OUT_CH = 128
EPS = 1e-5

NC = 2
NS = 16
NW = NC * NS
K = 128
NSPLIT = 1
N_ACC = 10240
STRIPE = N_ACC // NS


def _sc_mesh():
    return plsc.VectorSubcoreMesh(
        core_axis_name="c", subcore_axis_name="s", num_cores=NC, num_subcores=NS
    )


def _deg_body(na, nb, dst_hbm, zeros_hbm, deg_hbm, dst_v, zbuf, ones_v,
              obuf, deg_s):
    cid = lax.axis_index("c")
    sid = lax.axis_index("s")
    wid = cid * NS + sid
    nch = jnp.where(cid == 0, na, nb)

    pltpu.sync_copy(dst_hbm.at[wid], dst_v)
    pltpu.sync_copy(zeros_hbm, zbuf)
    pltpu.sync_copy(zbuf, deg_s.at[pl.ds(sid * STRIPE, STRIPE)])
    for i in range(K // 16):
        ones_v[pl.ds(i * 16, 16)] = jnp.ones((16,), jnp.float32)
    plsc.subcore_barrier()

    def chunk(j, carry):
        pltpu.sync_copy(ones_v, deg_s.at[dst_v.at[j]], add=True)
        return carry

    lax.fori_loop(0, nch, chunk, 0)
    plsc.subcore_barrier()

    pltpu.sync_copy(deg_s.at[pl.ds(sid * STRIPE, STRIPE)], obuf)
    pltpu.sync_copy(obuf, deg_hbm.at[cid, pl.ds(sid * STRIPE, STRIPE)])


def _deg_kernel(dst3, zeros1d, na, nb):
    D = dst3.shape[1]
    return pl.kernel(
        functools.partial(_deg_body, na, nb),
        out_type=jax.ShapeDtypeStruct((NC, N_ACC), jnp.float32),
        mesh=_sc_mesh(),
        scratch_types=[
            pltpu.VMEM((D, K), jnp.int32),
            pltpu.VMEM((STRIPE,), jnp.float32),
            pltpu.VMEM((K,), jnp.float32),
            pltpu.VMEM((STRIPE,), jnp.float32),
            pltpu.VMEM_SHARED((N_ACC,), jnp.float32),
        ],
    )(dst3, zeros1d)


def _g_body(x_ref, w_ref, deg_ref, g_ref):
    h = lax.dot_general(
        x_ref[...], w_ref[...], (((1,), (1,)), ((), ())),
        preferred_element_type=jnp.float32)
    deg = 1.0 + deg_ref[0, :] + deg_ref[1, :]
    dinv = lax.rsqrt(deg)
    g_ref[...] = h * dinv[:, None]


def _g_kernel(x_pad, W, deg2):
    nb = 10
    blk = N_ACC // nb
    return pl.pallas_call(
        _g_body,
        grid=(nb,),
        in_specs=[
            pl.BlockSpec((blk, IN_CH), lambda i: (i, 0)),
            pl.BlockSpec((OUT_CH, IN_CH), lambda i: (0, 0)),
            pl.BlockSpec((NC, blk), lambda i: (0, i)),
        ],
        out_specs=pl.BlockSpec((blk, OUT_CH), lambda i: (i, 0)),
        out_shape=jax.ShapeDtypeStruct((N_ACC, OUT_CH), jnp.float32),
    )(x_pad, W, deg2)


def _scatter_body(na, nb, g_hbm, src_hbm, dst_hbm, zeros_hbm, p_hbm,
                  sidx0, sidx1, didx0, didx1, rows0, rows1, acc_s,
                  sems_g0, sems_g1, sem_s0, sem_s1, sem_d0, sem_d1):
    cid = lax.axis_index("c")
    sid = lax.axis_index("s")
    wid = cid * NS + sid
    nch = jnp.where(cid == 0, na, nb)
    src_w = src_hbm.at[wid]
    dst_w = dst_hbm.at[wid]

    def idx_start(hbm_row, buf, sem):
        pltpu.async_copy(hbm_row, buf, sem)

    def idx_wait(hbm_row, buf, sem):
        pltpu.make_async_copy(hbm_row, buf, sem).wait()

    SLICE = K // NSPLIT

    def g_start(sbuf, buf, sems):
        for s in range(NSPLIT):
            pltpu.async_copy(g_hbm.at[sbuf.at[pl.ds(s * SLICE, SLICE)]],
                             buf.at[pl.ds(s * SLICE, SLICE)], sems[s])

    def g_wait(sbuf, buf, sems):
        for s in range(NSPLIT):
            pltpu.make_async_copy(g_hbm.at[sbuf.at[pl.ds(s * SLICE, SLICE)]],
                                  buf.at[pl.ds(s * SLICE, SLICE)],
                                  sems[s]).wait()

    idx_start(src_w.at[0], sidx0, sem_s0)
    idx_start(src_w.at[1], sidx1, sem_s1)
    idx_start(dst_w.at[0], didx0, sem_d0)
    idx_start(dst_w.at[1], didx1, sem_d1)

    pltpu.sync_copy(zeros_hbm, rows0)
    for k in range(STRIPE // K):
        pltpu.sync_copy(rows0, acc_s.at[pl.ds(sid * STRIPE + k * K, K)])
    plsc.subcore_barrier()

    idx_wait(src_w.at[0], sidx0, sem_s0)
    g_start(sidx0, rows0, sems_g0)
    idx_wait(src_w.at[1], sidx1, sem_s1)
    g_start(sidx1, rows1, sems_g1)

    def step(i, carry):
        c0 = 2 * i
        c1 = c0 + 1
        g_wait(sidx0, rows0, sems_g0)
        idx_start(src_w.at[c0 + 2], sidx0, sem_s0)
        idx_wait(dst_w.at[c0], didx0, sem_d0)
        pltpu.sync_copy(rows0, acc_s.at[didx0], add=True)
        idx_wait(src_w.at[c0 + 2], sidx0, sem_s0)
        g_start(sidx0, rows0, sems_g0)
        idx_start(dst_w.at[c0 + 2], didx0, sem_d0)
        g_wait(sidx1, rows1, sems_g1)
        idx_start(src_w.at[c1 + 2], sidx1, sem_s1)
        idx_wait(dst_w.at[c1], didx1, sem_d1)
        pltpu.sync_copy(rows1, acc_s.at[didx1], add=True)
        idx_wait(src_w.at[c1 + 2], sidx1, sem_s1)
        g_start(sidx1, rows1, sems_g1)
        idx_start(dst_w.at[c1 + 2], didx1, sem_d1)
        return carry

    lax.fori_loop(0, nch // 2, step, 0)
    g_wait(sidx0, rows0, sems_g0)
    g_wait(sidx1, rows1, sems_g1)
    idx_wait(dst_w.at[0], didx0, sem_d0)
    idx_wait(dst_w.at[1], didx1, sem_d1)
    plsc.subcore_barrier()

    for k in range(STRIPE // K):
        r0 = sid * STRIPE + k * K
        pltpu.sync_copy(acc_s.at[pl.ds(r0, K)], rows0)
        pltpu.sync_copy(rows0, p_hbm.at[cid, pl.ds(r0, K)])


def _scatter_kernel(g, src3, dst3, zeros2d, na, nb):
    return pl.kernel(
        functools.partial(_scatter_body, na, nb),
        out_type=jax.ShapeDtypeStruct((NC, N_ACC, OUT_CH), jnp.float32),
        mesh=_sc_mesh(),
        scratch_types=[
            pltpu.VMEM((K,), jnp.int32),
            pltpu.VMEM((K,), jnp.int32),
            pltpu.VMEM((K,), jnp.int32),
            pltpu.VMEM((K,), jnp.int32),
            pltpu.VMEM((K, OUT_CH), jnp.float32),
            pltpu.VMEM((K, OUT_CH), jnp.float32),
            pltpu.VMEM_SHARED((N_ACC, OUT_CH), jnp.float32),
            [pltpu.SemaphoreType.DMA] * NSPLIT,
            [pltpu.SemaphoreType.DMA] * NSPLIT,
            pltpu.SemaphoreType.DMA,
            pltpu.SemaphoreType.DMA,
            pltpu.SemaphoreType.DMA,
            pltpu.SemaphoreType.DMA,
        ],
    )(g, src3, dst3, zeros2d)


def _z_body(blk, p_ref, g_ref, deg_ref, b_ref, z_ref, stats_ref):
    i = pl.program_id(0)
    deg = 1.0 + deg_ref[0, :] + deg_ref[1, :]
    dinv = lax.rsqrt(deg)
    z = (p_ref[0] + p_ref[1] + g_ref[...]) * dinv[:, None] + b_ref[...]
    row = i * blk + lax.broadcasted_iota(jnp.int32, (blk, OUT_CH), 0)
    z = jnp.where(row < N_NODES, z, 0.0)
    z_ref[...] = z

    @pl.when(i == 0)
    def _():
        stats_ref[...] = jnp.zeros_like(stats_ref)

    s1 = jnp.sum(z, axis=0)[None, :]
    s2 = jnp.sum(z * z, axis=0)[None, :]
    pad = jnp.zeros((6, OUT_CH), jnp.float32)
    stats_ref[...] += jnp.concatenate([s1, s2, pad], axis=0)


def _z_kernel(P, g, deg2, b):
    nb = 10
    blk = N_ACC // nb
    return pl.pallas_call(
        functools.partial(_z_body, blk),
        grid=(nb,),
        in_specs=[
            pl.BlockSpec((NC, blk, OUT_CH), lambda i: (0, i, 0)),
            pl.BlockSpec((blk, OUT_CH), lambda i: (i, 0)),
            pl.BlockSpec((NC, blk), lambda i: (0, i)),
            pl.BlockSpec((1, OUT_CH), lambda i: (0, 0)),
        ],
        out_specs=[
            pl.BlockSpec((blk, OUT_CH), lambda i: (i, 0)),
            pl.BlockSpec((8, OUT_CH), lambda i: (0, 0)),
        ],
        out_shape=[
            jax.ShapeDtypeStruct((N_ACC, OUT_CH), jnp.float32),
            jax.ShapeDtypeStruct((8, OUT_CH), jnp.float32),
        ],
    )(P, g, deg2, b)


def _bn_body(z_ref, stats_ref, gamma_ref, beta_ref, o_ref):
    n = jnp.float32(N_NODES)
    mean = stats_ref[0, :] / n
    var = stats_ref[1, :] / n - mean * mean
    scale = gamma_ref[0, :] * lax.rsqrt(var + EPS)
    shift = beta_ref[0, :] - mean * scale
    o_ref[...] = jnp.maximum(z_ref[...] * scale[None, :] + shift[None, :], 0.0)


def _bn_kernel(z, stats, gamma, beta):
    nb = 10
    blk = N_ACC // nb
    return pl.pallas_call(
        _bn_body,
        grid=(nb,),
        in_specs=[
            pl.BlockSpec((blk, OUT_CH), lambda i: (i, 0)),
            pl.BlockSpec((8, OUT_CH), lambda i: (0, 0)),
            pl.BlockSpec((1, OUT_CH), lambda i: (0, 0)),
            pl.BlockSpec((1, OUT_CH), lambda i: (0, 0)),
        ],
        out_specs=pl.BlockSpec((blk, OUT_CH), lambda i: (i, 0)),
        out_shape=jax.ShapeDtypeStruct((N_ACC, OUT_CH), jnp.float32),
    )(z, stats, gamma, beta)


def kernel(x, edge_index, W, b, gamma, beta):
    E = edge_index.shape[1]
    src = edge_index[0].astype(jnp.int32)
    dst = edge_index[1].astype(jnp.int32)

    tot = -(-E // (NS * K))
    na = -(-(tot * 63) // 100)
    na += na % 2
    nb = tot - na
    nb += nb % 2
    cap = NS * (na + nb) * K
    src_p = jnp.concatenate([src, jnp.zeros((cap - E,), jnp.int32)])
    trash = N_NODES + jnp.arange(cap - E, dtype=jnp.int32) % (N_ACC - N_NODES)
    dst_p = jnp.concatenate([dst, trash])
    D = -(-(max(na, nb) + 2) // 8) * 8
    cut = NS * na * K

    def to3(flat, n, fill):
        part = flat.reshape(NS, n, K)
        pad = jnp.full((NS, D - n, K), fill, jnp.int32)
        return jnp.concatenate([part, pad], axis=1)

    src3 = jnp.concatenate(
        [to3(src_p[:cut], na, 0), to3(src_p[cut:], nb, 0)], axis=0)
    dst3 = jnp.concatenate(
        [to3(dst_p[:cut], na, N_NODES), to3(dst_p[cut:], nb, N_NODES)],
        axis=0)

    zeros1d = jnp.zeros((STRIPE,), jnp.float32)
    zeros2d = jnp.zeros((K, OUT_CH), jnp.float32)
    x_pad = jnp.concatenate(
        [x, jnp.zeros((N_ACC - N_NODES, IN_CH), jnp.float32)])

    deg2 = _deg_kernel(dst3, zeros1d, na, nb)
    g = _g_kernel(x_pad, W, deg2)
    P = _scatter_kernel(g, src3, dst3, zeros2d, na, nb)
    z, stats = _z_kernel(P, g, deg2, b.reshape(1, OUT_CH))
    out = _bn_kernel(z, stats, gamma.reshape(1, OUT_CH),
                     beta.reshape(1, OUT_CH))
    return out[:N_NODES]
